# baseline (speedup 1.0000x reference)
import functools

import jax
import jax.numpy as jnp
from jax import lax
from jax.experimental import pallas as pl
from jax.experimental.pallas import tpu as pltpu
from jax.experimental.pallas import tpu_sc as plsc

_B = 16384
_NCAT = 26
_NCONT = 13
_D = 64
_NTOK = 999987
_NPAIR = (_NTOK - 1) // 2
_CHUNK = 128
_NC = 2
_NS = 16
_NW = _NC * _NS
_NCHUNK = _NCAT * (_B // _CHUNK)
_CPW = _NCHUNK // _NW
_BPC = _B // _CHUNK

_mesh = plsc.VectorSubcoreMesh(core_axis_name="c", subcore_axis_name="s")


@functools.partial(
    pl.kernel,
    out_type=jax.ShapeDtypeStruct((_NCAT * _D, _B), jnp.float32),
    mesh=_mesh,
    compiler_params=pltpu.CompilerParams(
        needs_layout_passes=False, use_tc_tiling_on_sc=True),
    scratch_types=[
        pltpu.VMEM((_CPW, _CHUNK), jnp.int32),
        [pltpu.VMEM((_CHUNK, 2 * _D), jnp.float32)] * 2,
        [pltpu.VMEM((_D, _CHUNK), jnp.float32)] * 2,
        pltpu.SemaphoreType.DMA,
        [pltpu.SemaphoreType.DMA] * 2,
        [pltpu.SemaphoreType.DMA] * 2,
    ],
)
def _sc_gather(table_hbm, idx_hbm, out_hbm, idx_v, pbufs, tbufs, isem,
               gsems, wsems):
    wid = lax.axis_index("s") * _NC + lax.axis_index("c")
    base = wid * _CPW
    iota16 = lax.iota(jnp.int32, 16)

    for j in range(_CPW):
        p = base + j
        pltpu.async_copy(
            idx_hbm.at[p // _BPC, pl.ds((p % _BPC) * _CHUNK, _CHUNK)],
            idx_v.at[j], isem)
    pltpu.make_async_copy(idx_hbm.at[pl.ds(0, 1), pl.ds(0, _CHUNK)], idx_v,
                          isem).wait()

    def issue_gathers(j, b):
        for g in range(8):
            iv = idx_v[j, pl.ds(g * 16, 16)]
            pltpu.async_copy(table_hbm.at[iv >> 1],
                             pbufs[b].at[pl.ds(g * 16, 16)], gsems[b])

    def drain_gather(b):
        pltpu.make_async_copy(table_hbm.at[idx_v.at[0]], pbufs[b],
                              gsems[b]).wait()

    def transpose_chunk(j, b):
        hvs = [(idx_v[j, pl.ds(g * 16, 16)] & 1) * _D for g in range(8)]
        rows = [iota16 + g * 16 for g in range(8)]
        pb, tb = pbufs[b], tbufs[b]

        @plsc.parallel_loop(0, _D, unroll=4)
        def dbody(d):
            for g in range(8):
                val = plsc.load_gather(pb, [rows[g], hvs[g] + d])
                tb[d, pl.ds(g * 16, 16)] = val

    def issue_write(j, b):
        p = base + j
        pltpu.async_copy(
            tbufs[b],
            out_hbm.at[pl.ds((p // _BPC) * _D, _D),
                       pl.ds((p % _BPC) * _CHUNK, _CHUNK)],
            wsems[b])

    def drain_write(b):
        pltpu.make_async_copy(
            tbufs[b], out_hbm.at[pl.ds(0, _D), pl.ds(0, _CHUNK)],
            wsems[b]).wait()

    issue_gathers(0, 0)
    issue_gathers(1, 1)
    for b in range(2):
        drain_gather(b)
        transpose_chunk(b, b)
        issue_write(b, b)
        issue_gathers(2 + b, b)

    def body(g, carry):
        for b in range(2):
            j = 2 * g + b
            drain_write(b)
            drain_gather(b)
            transpose_chunk(j, b)
            issue_write(j, b)
            issue_gathers(j + 2, b)
        return carry

    lax.fori_loop(1, _CPW // 2 - 1, body, 0)

    for b in range(2):
        j = _CPW - 2 + b
        drain_write(b)
        drain_gather(b)
        transpose_chunk(j, b)
        issue_write(j, b)
    for b in range(2):
        drain_write(b)


_CB = 2048


def _pair_body(x_ref, o_ref):
    x = x_ref[...]
    y = x.reshape(_D, _CB // 2, 2)
    o_ref[...] = y.transpose(1, 2, 0).reshape(_CB // 2, 2 * _D)


def _pair_table(table_t):
    return pl.pallas_call(
        _pair_body,
        out_shape=jax.ShapeDtypeStruct((_NPAIR, 2 * _D), jnp.float32),
        grid=(pl.cdiv(_NPAIR, _CB // 2),),
        in_specs=[pl.BlockSpec((_D, _CB), lambda i: (0, i))],
        out_specs=pl.BlockSpec((_CB // 2, 2 * _D), lambda i: (i, 0)),
    )(table_t)


def _cont_body(w_ref, x_ref, b_ref, o_ref):
    o_ref[...] = (
        jnp.dot(w_ref[...], x_ref[...], preferred_element_type=jnp.float32,
                precision=jax.lax.Precision.HIGHEST)
        + b_ref[...]
    )


_BB = 2048
_DF = _NCONT * _D


def _cont_embed(w2t, xct, b2t):
    return pl.pallas_call(
        _cont_body,
        out_shape=jax.ShapeDtypeStruct((_DF, _B), jnp.float32),
        grid=(_B // _BB,),
        in_specs=[
            pl.BlockSpec((_DF, _NCONT), lambda i: (0, 0)),
            pl.BlockSpec((_NCONT, _BB), lambda i: (0, i)),
            pl.BlockSpec((_DF, 1), lambda i: (0, 0)),
        ],
        out_specs=pl.BlockSpec((_DF, _BB), lambda i: (0, i)),
    )(w2t, xct, b2t)


def kernel(X, table, cont_w, cont_b):
    xt = X.T
    idx_t = xt[:_NCAT].astype(jnp.int32)
    xct = xt[_NCAT:_NCAT + _NCONT]
    table2 = _pair_table(table.T)
    w2t = (jnp.eye(_NCONT, dtype=jnp.float32)[:, :, None]
           * cont_w[None, :, :]).reshape(_NCONT, _DF).T
    b2t = cont_b.reshape(_DF)[:, None]

    cat2d = _sc_gather(table2, idx_t)
    cont2d = _cont_embed(w2t, xct, b2t)

    x_cat = cat2d.reshape(_NCAT, _D, _B).transpose(2, 0, 1)
    x_cont = cont2d.reshape(_NCONT, _D, _B).transpose(2, 0, 1)
    return x_cat, x_cont


# speedup vs baseline: 9.4855x; 9.4855x over previous
import functools

import jax
import jax.numpy as jnp
from jax import lax
from jax.experimental import pallas as pl
from jax.experimental.pallas import tpu as pltpu
from jax.experimental.pallas import tpu_sc as plsc

_B = 16384
_NCAT = 26
_NCONT = 13
_D = 64
_NTOK = 999987
_NPAIR = (_NTOK - 1) // 2
_CHUNK = 128
_NC = 2
_NS = 16
_NW = _NC * _NS
_NCHUNK = _NCAT * (_B // _CHUNK)
_CPW = _NCHUNK // _NW
_BPC = _B // _CHUNK

_mesh = plsc.VectorSubcoreMesh(core_axis_name="c", subcore_axis_name="s")


@functools.partial(
    pl.kernel,
    out_type=jax.ShapeDtypeStruct((_NCAT * _D, _B), jnp.float32),
    mesh=_mesh,
    compiler_params=pltpu.CompilerParams(
        needs_layout_passes=False, use_tc_tiling_on_sc=True),
    scratch_types=[
        pltpu.VMEM((_CPW, _CHUNK), jnp.int32),
        [pltpu.VMEM((_CHUNK, 2 * _D), jnp.float32)] * 2,
        [pltpu.VMEM((_D, _CHUNK), jnp.float32)] * 2,
        pltpu.SemaphoreType.DMA,
        [pltpu.SemaphoreType.DMA] * 2,
        [pltpu.SemaphoreType.DMA] * 2,
    ],
)
def _sc_gather(table_hbm, idx_hbm, out_hbm, idx_v, pbufs, tbufs, isem,
               gsems, wsems):
    wid = lax.axis_index("s") * _NC + lax.axis_index("c")
    base = wid * _CPW
    iota16 = lax.iota(jnp.int32, 16)

    for j in range(_CPW):
        p = base + j
        pltpu.async_copy(
            idx_hbm.at[p // _BPC, pl.ds((p % _BPC) * _CHUNK, _CHUNK)],
            idx_v.at[j], isem)
    pltpu.make_async_copy(idx_hbm.at[pl.ds(0, 1), pl.ds(0, _CHUNK)], idx_v,
                          isem).wait()

    def issue_gathers(j, b):
        for g in range(8):
            iv = idx_v[j, pl.ds(g * 16, 16)]
            pltpu.async_copy(table_hbm.at[iv >> 1],
                             pbufs[b].at[pl.ds(g * 16, 16)], gsems[b])

    def drain_gather(b):
        pltpu.make_async_copy(table_hbm.at[idx_v.at[0]], pbufs[b],
                              gsems[b]).wait()

    def transpose_chunk(j, b):
        hvs = [(idx_v[j, pl.ds(g * 16, 16)] & 1) * _D for g in range(8)]
        rows = [iota16 + g * 16 for g in range(8)]
        pb, tb = pbufs[b], tbufs[b]

        @plsc.parallel_loop(0, _D, unroll=4)
        def dbody(d):
            for g in range(8):
                val = plsc.load_gather(pb, [rows[g], hvs[g] + d])
                tb[d, pl.ds(g * 16, 16)] = val

    def issue_write(j, b):
        p = base + j
        pltpu.async_copy(
            tbufs[b],
            out_hbm.at[pl.ds((p // _BPC) * _D, _D),
                       pl.ds((p % _BPC) * _CHUNK, _CHUNK)],
            wsems[b])

    def drain_write(b):
        pltpu.make_async_copy(
            tbufs[b], out_hbm.at[pl.ds(0, _D), pl.ds(0, _CHUNK)],
            wsems[b]).wait()

    issue_gathers(0, 0)
    issue_gathers(1, 1)
    for b in range(2):
        drain_gather(b)
        transpose_chunk(b, b)
        issue_write(b, b)
        issue_gathers(2 + b, b)

    def body(g, carry):
        for b in range(2):
            j = 2 * g + b
            drain_write(b)
            drain_gather(b)
            transpose_chunk(j, b)
            issue_write(j, b)
            issue_gathers(j + 2, b)
        return carry

    lax.fori_loop(1, _CPW // 2 - 1, body, 0)

    for b in range(2):
        j = _CPW - 2 + b
        drain_write(b)
        drain_gather(b)
        transpose_chunk(j, b)
        issue_write(j, b)
    for b in range(2):
        drain_write(b)


def _cont_body(w_ref, x_ref, b_ref, o_ref):
    o_ref[...] = (
        jnp.dot(w_ref[...], x_ref[...], preferred_element_type=jnp.float32,
                precision=jax.lax.Precision.HIGHEST)
        + b_ref[...]
    )


_BB = 2048
_DF = _NCONT * _D


def _cont_embed(w2t, xct, b2t):
    return pl.pallas_call(
        _cont_body,
        out_shape=jax.ShapeDtypeStruct((_DF, _B), jnp.float32),
        grid=(_B // _BB,),
        in_specs=[
            pl.BlockSpec((_DF, _NCONT), lambda i: (0, 0)),
            pl.BlockSpec((_NCONT, _BB), lambda i: (0, i)),
            pl.BlockSpec((_DF, 1), lambda i: (0, 0)),
        ],
        out_specs=pl.BlockSpec((_DF, _BB), lambda i: (0, i)),
    )(w2t, xct, b2t)


def kernel(X, table, cont_w, cont_b):
    xt = X.T
    idx_t = xt[:_NCAT].astype(jnp.int32)
    xct = xt[_NCAT:_NCAT + _NCONT]
    table2 = table[:_NTOK - 1].reshape(_NPAIR, 2 * _D)
    w2t = (jnp.eye(_NCONT, dtype=jnp.float32)[:, :, None]
           * cont_w[None, :, :]).reshape(_NCONT, _DF).T
    b2t = cont_b.reshape(_DF)[:, None]

    cat2d = _sc_gather(table2, idx_t)
    cont2d = _cont_embed(w2t, xct, b2t)

    x_cat = cat2d.reshape(_NCAT, _D, _B).transpose(2, 0, 1)
    x_cont = cont2d.reshape(_NCONT, _D, _B).transpose(2, 0, 1)
    return x_cat, x_cont
